# Initial kernel scaffold; baseline (speedup 1.0000x reference)
import jax
import jax.numpy as jnp
from jax.experimental import pallas as pl


def kernel(label, directions):
    raise NotImplementedError("write your pallas kernel here")


# speedup vs baseline: 75.1108x; 75.1108x over previous
import functools

import numpy as np
import jax
import jax.numpy as jnp
from jax import lax
from jax.experimental import pallas as pl
from jax.experimental.pallas import tpu as pltpu

_SKIP = 10
_THR = 0.9
_NB = 30
_D = 20
_KW = 2 * _D + 1
_PERC = 0.5
_MAXO = 10
_EPS = 1e-8
_H, _W = 480, 640
_GH, _GW = _H // _SKIP, _W // _SKIP
_PH, _PW = (_H + 2 * _D) // _SKIP, (_W + 2 * _D) // _SKIP


def _build_offset_table():
    off = np.arange(-_D, _D + 1)
    oy, ox = np.meshgrid(off, off, indexing="ij")
    sq = (oy ** 2 + ox ** 2).astype(np.float32)
    dist = np.sqrt(sq)
    nrm = np.maximum(dist, np.float32(_EPS))
    uy = (-oy.astype(np.float32)) / nrm
    ux = (-ox.astype(np.float32)) / nrm
    ang = np.arctan2(uy, ux).astype(np.float32)
    x = (ang + np.float32(np.pi)) / np.float32(2.0 * np.pi) * np.float32(_NB)
    bins = np.clip(np.floor(x), 0, _NB - 1).astype(np.int32)
    valid = (dist <= _D) & (dist > 0)

    oyp = oy + _D
    oxp = ox + _D
    q = oyp // _SKIP
    r = oyp % _SKIP
    p = oxp // _SKIP
    s = oxp % _SKIP
    phase = r * _SKIP + s

    ints, flts, groups = [], [], []
    start = 0
    for qq in range(5):
        for pp in range(5):
            sel = valid & (q == qq) & (p == pp)
            if not sel.any():
                continue
            cnt = int(sel.sum())
            ints.append(np.stack([phase[sel],
                                  (1 << bins[sel]).astype(np.int32)], axis=1))
            flts.append(np.stack([uy[sel], ux[sel]], axis=1))
            groups.append((qq, pp, start, start + cnt))
            start += cnt
    return (np.concatenate(ints, axis=0).astype(np.int32).reshape(-1),
            np.concatenate(flts, axis=0).astype(np.float32).reshape(-1),
            tuple(groups))


_PI_TAB, _PF_TAB, _GROUPS = _build_offset_table()
_K = _PI_TAB.shape[0] // 2


def _vote_body(pi_ref, pf_ref, labq_ref, dyq_ref, dxq_ref,
               vals_ref, idx_ref, nobj_ref,
               uys_ref, uxs_ref, nms_ref):
    def _norm(i, _):
        dy = dyq_ref[0, i]
        dx = dxq_ref[0, i]
        dn = jnp.sqrt(dy * dy + dx * dx) + _EPS
        uys_ref[i] = dy / dn
        uxs_ref[i] = dx / dn
        return 0
    lax.fori_loop(0, _SKIP * _SKIP, _norm, 0, unroll=False)

    bits = jnp.zeros((_GH, _GW), jnp.int32)
    for (qq, pp, s0, s1) in _GROUPS:
        def _body(k, b, qq=qq, pp=pp):
            ph = pi_ref[2 * k]
            bit = pi_ref[2 * k + 1]
            uyf = pf_ref[2 * k]
            uxf = pf_ref[2 * k + 1]
            labw = labq_ref[0, ph, pl.ds(qq, _GH), pl.ds(pp, _GW)]
            uyw = uys_ref[ph, pl.ds(qq, _GH), pl.ds(pp, _GW)]
            uxw = uxs_ref[ph, pl.ds(qq, _GH), pl.ds(pp, _GW)]
            cos = uyw * uyf + uxw * uxf
            inl = (labw > 0.5) & (cos > _THR)
            return jnp.where(inl, b | bit, b)
        bits = lax.fori_loop(s0, s1, _body, bits)

    cnt = lax.population_count(bits).astype(jnp.float32)
    frac = cnt / float(_NB)
    fg = labq_ref[0, 0, pl.ds(2, _GH), pl.ds(2, _GW)] > 0.5
    score = jnp.where(fg, frac, 0.0)

    nms_ref[...] = jnp.zeros((_GH + 8, _GW + 8), jnp.float32)
    nms_ref[pl.ds(4, _GH), pl.ds(4, _GW)] = score
    lm = None
    for a in range(3):
        for b in range(3):
            sl = nms_ref[pl.ds(3 + a, _GH), pl.ds(3 + b, _GW)]
            lm = sl if lm is None else jnp.maximum(lm, sl)
    peak = (score >= _PERC) & (score >= lm)
    pk = jnp.where(peak, score, 0.0)

    ri = lax.broadcasted_iota(jnp.int32, (_GH, _GW), 0)
    ci = lax.broadcasted_iota(jnp.int32, (_GH, _GW), 1)
    flat = ri * _GW + ci

    def _tk(k, st):
        t, zc, seen00, cnt_obj = st
        m = jnp.max(t)
        g = jnp.min(jnp.where(t == m, flat, _GH * _GW))
        full = (g // _GW) * (_W * _SKIP) + (g % _GW) * _SKIP
        pos = m > 0.0
        seen00 = seen00 | (pos & (full == 0))
        fill = zc + seen00.astype(jnp.int32)
        vals_ref[0, 0, k] = m
        idx_ref[0, 0, k] = jnp.where(pos, full, fill)
        t = jnp.where(flat == g, -1.0, t)
        return (t, zc + 1 - pos.astype(jnp.int32), seen00,
                cnt_obj + pos.astype(jnp.int32))

    _, _, _, cnt_obj = lax.fori_loop(0, _MAXO, _tk,
                                     (pk, 0, False, 0))
    nobj_ref[0, 0, 0] = cnt_obj


_ROWS_B = 48


def _mask_body(idx_ref, vals_ref, labf_ref, dy_ref, dx_ref, out_ref):
    n = pl.program_id(0)
    rt = pl.program_id(1)
    dy = dy_ref[0]
    dx = dx_ref[0]
    dn = jnp.sqrt(dy * dy + dx * dx) + _EPS
    uy = dy / dn
    ux = dx / dn
    fg = labf_ref[0] > 0.5
    py = (rt * _ROWS_B
          + lax.broadcasted_iota(jnp.int32, (_ROWS_B, _W), 0)).astype(jnp.float32)
    px = lax.broadcasted_iota(jnp.int32, (_ROWS_B, _W), 1).astype(jnp.float32)
    acc = jnp.zeros((_ROWS_B, _W), jnp.float32)
    for m in range(_MAXO):
        ti = idx_ref[n, 0, m]
        cy = (ti // _W).astype(jnp.float32)
        cx = (ti % _W).astype(jnp.float32)
        valid = vals_ref[n, 0, m] > 0.0
        vy = cy - py
        vx = cx - px
        nn = jnp.sqrt(vy * vy + vx * vx) + _EPS
        cos2 = (uy * vy + ux * vx) / nn
        ind = fg & (cos2 > _THR) & valid
        acc = acc + jnp.where(ind, float(m + 2), 0.0)
    out_ref[0] = acc


def _phase_split(x):
    n = x.shape[0]
    x = x.reshape(n, _PH, _SKIP, _PW, _SKIP)
    x = x.transpose(0, 2, 4, 1, 3)
    return x.reshape(n, _SKIP * _SKIP, _PH, _PW)


def kernel(label, directions):
    n = label.shape[0]
    label_f = label.astype(jnp.float32)
    dy = directions[:, 0]
    dx = directions[:, 1]
    pad = ((0, 0), (_D, _D), (_D, _D))
    labq = _phase_split(jnp.pad(label_f, pad))
    dyq = _phase_split(jnp.pad(dy, pad))
    dxq = _phase_split(jnp.pad(dx, pad))
    pi_tab = jnp.asarray(_PI_TAB)
    pf_tab = jnp.asarray(_PF_TAB)

    smem = functools.partial(pl.BlockSpec, memory_space=pltpu.SMEM)
    vals, idx, nobj = pl.pallas_call(
        _vote_body,
        grid=(n,),
        in_specs=[
            smem((2 * _K,), lambda i: (0,)),
            smem((2 * _K,), lambda i: (0,)),
            pl.BlockSpec((1, _SKIP * _SKIP, _PH, _PW), lambda i: (i, 0, 0, 0)),
            pl.BlockSpec((1, _SKIP * _SKIP, _PH, _PW), lambda i: (i, 0, 0, 0)),
            pl.BlockSpec((1, _SKIP * _SKIP, _PH, _PW), lambda i: (i, 0, 0, 0)),
        ],
        out_specs=[
            smem((1, 1, _MAXO), lambda i: (i, 0, 0)),
            smem((1, 1, _MAXO), lambda i: (i, 0, 0)),
            smem((1, 1, 1), lambda i: (i, 0, 0)),
        ],
        out_shape=[
            jax.ShapeDtypeStruct((n, 1, _MAXO), jnp.float32),
            jax.ShapeDtypeStruct((n, 1, _MAXO), jnp.int32),
            jax.ShapeDtypeStruct((n, 1, 1), jnp.int32),
        ],
        scratch_shapes=[
            pltpu.VMEM((_SKIP * _SKIP, _PH, _PW), jnp.float32),
            pltpu.VMEM((_SKIP * _SKIP, _PH, _PW), jnp.float32),
            pltpu.VMEM((_GH + 8, _GW + 8), jnp.float32),
        ],
    )(pi_tab, pf_tab, labq, dyq, dxq)

    masks = pl.pallas_call(
        _mask_body,
        grid=(n, _H // _ROWS_B),
        in_specs=[
            smem((n, 1, _MAXO), lambda i, j: (0, 0, 0)),
            smem((n, 1, _MAXO), lambda i, j: (0, 0, 0)),
            pl.BlockSpec((1, _ROWS_B, _W), lambda i, j: (i, j, 0)),
            pl.BlockSpec((1, _ROWS_B, _W), lambda i, j: (i, j, 0)),
            pl.BlockSpec((1, _ROWS_B, _W), lambda i, j: (i, j, 0)),
        ],
        out_specs=pl.BlockSpec((1, _ROWS_B, _W), lambda i, j: (i, j, 0)),
        out_shape=jax.ShapeDtypeStruct((n, _H, _W), jnp.float32),
    )(idx, vals, label_f, dy, dx)

    idx2 = idx.reshape(n, _MAXO)
    centers = jnp.stack([idx2 / _W, (idx2 % _W).astype(jnp.float32)], axis=1)
    return masks, nobj.reshape(n), centers
